# baseline (speedup 1.0000x reference)
import functools

import jax
import jax.numpy as jnp
from jax import lax
from jax.experimental import pallas as pl
from jax.experimental.pallas import tpu as pltpu
from jax.experimental.pallas import tpu_sc as plsc

_BM = 2048


def _sc_row_gather(table, idx):
    h = table.shape[1]
    mesh = plsc.VectorSubcoreMesh(core_axis_name="c", subcore_axis_name="s")

    @functools.partial(
        pl.kernel,
        mesh=mesh,
        out_type=jax.ShapeDtypeStruct((1, h), jnp.float32),
        scratch_types=[
            pltpu.VMEM((1,), jnp.int32),
            pltpu.VMEM((1, h), jnp.float32),
            pltpu.SemaphoreType.DMA,
        ],
    )
    def k(table_hbm, idx_hbm, out_hbm, idx_v, row_v, sem):
        wid = lax.axis_index("s") * 2 + lax.axis_index("c")

        @pl.when(wid == 0)
        def _():
            pltpu.sync_copy(idx_hbm, idx_v)
            pltpu.async_copy(table_hbm.at[idx_v], row_v, sem).wait()
            pltpu.sync_copy(row_v, out_hbm)

    return k(table, idx)


def _add_row_kernel(row_ref, x_ref, o_ref):
    o_ref[...] = x_ref[...] + row_ref[...]


def kernel(embeds, embed_type, table):
    b, s, h = embeds.shape
    n = b * s
    x = embeds.reshape(n, h)
    idx = jnp.asarray(embed_type, dtype=jnp.int32).reshape(1)
    row = _sc_row_gather(table, idx)
    out = pl.pallas_call(
        _add_row_kernel,
        grid=(n // _BM,),
        in_specs=[
            pl.BlockSpec((1, h), lambda i: (0, 0)),
            pl.BlockSpec((_BM, h), lambda i: (i, 0)),
        ],
        out_specs=pl.BlockSpec((_BM, h), lambda i: (i, 0)),
        out_shape=jax.ShapeDtypeStruct((n, h), embeds.dtype),
        compiler_params=pltpu.CompilerParams(
            dimension_semantics=("parallel",),
        ),
    )(row, x)
    return out.reshape(b, s, h)


# speedup vs baseline: 1.4989x; 1.4989x over previous
import jax
import jax.numpy as jnp
from jax.experimental import pallas as pl
from jax.experimental.pallas import tpu as pltpu

_BM = 3584


def _add_row_kernel(idx_ref, table_ref, x_ref, o_ref):
    row = table_ref[idx_ref[0], :]
    o_ref[...] = x_ref[...] + row[None, :]


def kernel(embeds, embed_type, table):
    b, s, h = embeds.shape
    n = b * s
    x = embeds.reshape(n, h)
    idx = jnp.asarray(embed_type, dtype=jnp.int32).reshape(1)
    out = pl.pallas_call(
        _add_row_kernel,
        grid_spec=pltpu.PrefetchScalarGridSpec(
            num_scalar_prefetch=1,
            grid=(pl.cdiv(n, _BM),),
            in_specs=[
                pl.BlockSpec(table.shape, lambda i, idx_ref: (0, 0)),
                pl.BlockSpec((_BM, h), lambda i, idx_ref: (i, 0)),
            ],
            out_specs=pl.BlockSpec((_BM, h), lambda i, idx_ref: (i, 0)),
        ),
        out_shape=jax.ShapeDtypeStruct((n, h), embeds.dtype),
        compiler_params=pltpu.CompilerParams(
            dimension_semantics=("parallel",),
        ),
    )(idx, table, x)
    return out.reshape(b, s, h)
